# baseline (speedup 1.0000x reference)
import functools

import jax
import jax.numpy as jnp
from jax import lax
from jax.experimental import pallas as pl
from jax.experimental.pallas import tpu as pltpu
from jax.experimental.pallas import tpu_sc as plsc

EPS = 1e-12
LANES = 16
RQ = 4
JU = 6


def _sc_embed_ln(xf, ttf, token_emb, pos_emb, type_emb, ln_gamma, ln_beta,
                 *, n_tokens, seq, hid):
    NC, NS = 2, 16
    NW = NC * NS
    tpw = n_tokens // NW
    CS = 32
    spc = seq // CS
    bpw = tpw // seq
    n_units = spc * bpw
    JD = hid // LANES
    n_types = type_emb.shape[0]

    mesh = plsc.VectorSubcoreMesh(core_axis_name="c", subcore_axis_name="s")

    @functools.partial(
        pl.kernel,
        out_type=jax.ShapeDtypeStruct((n_tokens, hid), jnp.float32),
        mesh=mesh,
        scratch_types=[
            pltpu.VMEM((CS,), jnp.int32),
            pltpu.VMEM((CS,), jnp.int32),
            pltpu.VMEM((CS,), jnp.int32),
            pltpu.VMEM((CS,), jnp.int32),
            pltpu.VMEM((CS, hid), jnp.float32),
            pltpu.VMEM((CS, hid), jnp.float32),
            pltpu.VMEM((CS, hid), jnp.float32),
            pltpu.VMEM((n_types, hid), jnp.float32),
            pltpu.VMEM((hid,), jnp.float32),
            pltpu.VMEM((hid,), jnp.float32),
            pltpu.VMEM((hid,), jnp.float32),
            pltpu.VMEM((CS, LANES), jnp.float32),
            pltpu.VMEM((CS, LANES), jnp.float32),
            pltpu.SemaphoreType.DMA,
            pltpu.SemaphoreType.DMA,
            pltpu.SemaphoreType.DMA,
            pltpu.SemaphoreType.DMA,
        ],
    )
    def k(x_h, tt_h, tok_h, pos_h, typ_h, g_h, b_h, out_h,
          idx0, idx1, tt0, tt1, buf0, buf1, pbuf, typ_v, dt_v, g_v, b_v,
          mean_s, rstd_s, gsem0, gsem1, wsem0, wsem1):
        idx = (idx0, idx1)
        tts = (tt0, tt1)
        buf = (buf0, buf1)
        gsem = (gsem0, gsem1)
        wsem = (wsem0, wsem1)

        wid = lax.axis_index("s") * NC + lax.axis_index("c")
        base = wid * tpw
        pltpu.sync_copy(g_h, g_v)
        pltpu.sync_copy(b_h, b_v)
        pltpu.sync_copy(typ_h, typ_v)
        for j in range(JD):
            sl = pl.ds(j * LANES, LANES)
            dt_v[sl] = typ_v[1, sl] - typ_v[0, sl]

        inv_d = jnp.float32(1.0 / hid)
        dnums = lax.GatherDimensionNumbers(
            offset_dims=(), collapsed_slice_dims=(0,), start_index_map=(0,))

        def dyn_gather(v, perm):
            return lax.gather(
                v, perm[:, None], dnums, slice_sizes=(1,),
                mode=lax.GatherScatterMode.PROMISE_IN_BOUNDS)

        def splat(v, lane):
            return dyn_gather(v, lax.broadcast(lane, (LANES,)))

        def lane_sum(v):
            for sh in (8, 4, 2, 1):
                perm = jnp.arange(LANES, dtype=jnp.int32) ^ jnp.int32(sh)
                v = v + dyn_gather(v, perm)
            return v

        def unit_g0(u):
            return base + lax.rem(u, bpw) * seq + (u // bpw) * CS

        def fetch(u, p):
            g0 = unit_g0(u)
            pltpu.sync_copy(x_h.at[pl.ds(g0, CS)], idx[p])
            pltpu.sync_copy(tt_h.at[pl.ds(g0, CS)], tts[p])
            pltpu.async_copy(tok_h.at[idx[p]], buf[p], gsem[p])

        def load_pbuf(u):
            s0 = (u // bpw) * CS
            pltpu.sync_copy(pos_h.at[pl.ds(s0, CS)], pbuf)

        def compute(p):
            bp = buf[p]
            ttp = tts[p]

            def quad1_body(q, carry):
                r0 = q * RQ
                blk = (q // (LANES // RQ)) * LANES
                t16 = ttp[pl.ds(blk, LANES)].astype(jnp.float32)
                lane0 = lax.rem(q, LANES // RQ) * RQ
                tf = [splat(t16, lane0 + i) for i in range(RQ)]

                def jblk1(jj, carry):
                    accs, acc2s = carry
                    accs, acc2s = list(accs), list(acc2s)
                    for jo in range(JU):
                        sl = pl.ds(jj * (JU * LANES) + jo * LANES, LANES)
                        t0j = typ_v[0, sl]
                        dtj = dt_v[sl]
                        for i in range(RQ):
                            r = r0 + i
                            v = (bp[r, sl] + t0j) + (pbuf[r, sl]
                                                     + tf[i] * dtj)
                            bp[r, sl] = v
                            accs[i] = accs[i] + v
                            acc2s[i] = acc2s[i] + v * v
                    return tuple(accs), tuple(acc2s)

                zero = jnp.zeros((LANES,), jnp.float32)
                accs, acc2s = lax.fori_loop(
                    0, JD // JU, jblk1,
                    (tuple([zero] * RQ), tuple([zero] * RQ)), unroll=False)

                for i in range(RQ):
                    m = lane_sum(accs[i]) * inv_d
                    ex = lane_sum(acc2s[i]) * inv_d - m * m + jnp.float32(EPS)
                    xi = lax.bitcast_convert_type(ex, jnp.int32)
                    yi = jnp.int32(0x5F3759DF) - lax.shift_right_arithmetic(
                        xi, jnp.int32(1))
                    y = lax.bitcast_convert_type(yi, jnp.float32)
                    for _ in range(3):
                        y = y * (jnp.float32(1.5)
                                 - jnp.float32(0.5) * ex * y * y)
                    mean_s[r0 + i, :] = m
                    rstd_s[r0 + i, :] = y
                return carry

            lax.fori_loop(0, CS // RQ, quad1_body, 0, unroll=False)

            def quad2_body(q, carry):
                r0 = q * RQ
                ms = [mean_s[r0 + i, :] for i in range(RQ)]
                ys = [rstd_s[r0 + i, :] for i in range(RQ)]

                def jblk2(jj, carry):
                    for jo in range(JU):
                        sl = pl.ds(jj * (JU * LANES) + jo * LANES, LANES)
                        gj = g_v[sl]
                        bj = b_v[sl]
                        for i in range(RQ):
                            r = r0 + i
                            bp[r, sl] = (bp[r, sl] - ms[i]) * ys[i] * gj + bj
                    return carry

                return lax.fori_loop(0, JD // JU, jblk2, carry, unroll=False)

            lax.fori_loop(0, CS // RQ, quad2_body, 0, unroll=False)

        def write(u, p):
            pltpu.async_copy(buf[p], out_h.at[pl.ds(unit_g0(u), CS)], wsem[p])

        def unit_step(u, p):
            @pl.when(u >= 1)
            def _():
                pltpu.make_async_copy(
                    buf[1 - p], out_h.at[pl.ds(0, CS)], wsem[1 - p]).wait()

            @pl.when(u < n_units - 1)
            def _():
                fetch(u + 1, 1 - p)

            @pl.when(lax.rem(u, bpw) == 0)
            def _():
                load_pbuf(u)

            pltpu.make_async_copy(tok_h.at[idx[p]], buf[p], gsem[p]).wait()
            compute(p)
            write(u, p)

        fetch(jnp.int32(0), 0)

        def unit_body(u, carry):
            lax.cond(lax.rem(u, 2) == 0,
                     lambda: unit_step(u, 0),
                     lambda: unit_step(u, 1))
            return carry

        lax.fori_loop(0, n_units, unit_body, 0, unroll=False)

        pltpu.make_async_copy(buf[1], out_h.at[pl.ds(0, CS)], wsem[1]).wait()

    return k(xf, ttf, token_emb, pos_emb, type_emb, ln_gamma, ln_beta)


def kernel(x, token_type_ids, token_emb, pos_emb, type_emb, ln_gamma, ln_beta):
    batch, seq = x.shape
    hid = token_emb.shape[1]
    n_tokens = batch * seq
    out = _sc_embed_ln(
        x.reshape(n_tokens), token_type_ids.reshape(n_tokens),
        token_emb, pos_emb, type_emb, ln_gamma, ln_beta,
        n_tokens=n_tokens, seq=seq, hid=hid)
    return out.reshape(batch, seq, hid)
